# baseline (speedup 1.0000x reference)
import math
import functools

import numpy as np
import jax
import jax.numpy as jnp
from jax import lax
from jax.experimental import pallas as pl
from jax.experimental.pallas import tpu as pltpu
from jax.experimental.pallas import tpu_sc as plsc

_GRID = 32
_MAX_FRAMES = 32
_EMBED_DIM = 384
_SPATIAL_DIM = _EMBED_DIM * 2 // 3
_TEMPORAL_DIM = _EMBED_DIM - _SPATIAL_DIM
_X_DIM = _SPATIAL_DIM // 2
_Y_DIM = _SPATIAL_DIM - _X_DIM
_N = _GRID * _GRID

_NW = 32
_RPW = _N // _NW


def _create_pe_np(max_len, d):
    pos = np.arange(max_len, dtype=np.float32)[:, None]
    pe = np.zeros((max_len, d), dtype=np.float32)
    num_even = (d + 1) // 2
    num_odd = d // 2
    div_even = np.exp(np.arange(num_even, dtype=np.float32) * 2.0 * (-math.log(10000.0) / d))
    pe[:, 0::2] = np.sin(pos * div_even)
    if num_odd > 0:
        div_odd = np.exp(np.arange(num_odd, dtype=np.float32) * 2.0 * (-math.log(10000.0) / d))
        pe[:, 1::2] = np.cos(pos * div_odd)
    return pe


@functools.lru_cache(maxsize=None)
def _pe_tables(T):
    pe_x_tab = _create_pe_np(_GRID, _X_DIM)
    pe_y_tab = _create_pe_np(_GRID, _Y_DIM)
    pe_t_tab = _create_pe_np(_MAX_FRAMES, _TEMPORAL_DIM)
    yy, xx = np.meshgrid(np.arange(_GRID), np.arange(_GRID), indexing="ij")
    pe_x = pe_x_tab[xx.flatten()]
    pe_y = pe_y_tab[yy.flatten()]
    pe_spatial = np.concatenate([pe_x, pe_y], axis=-1)
    pe_t = pe_t_tab[:T]
    return jnp.asarray(pe_spatial.reshape(-1)), jnp.asarray(pe_t.reshape(-1))


_NBUF = 4


@functools.lru_cache(maxsize=None)
def _make_sc_kernel(planes, T):
    E = _EMBED_DIM
    SD = _SPATIAL_DIM
    TD = _TEMPORAL_DIM
    chunk = _RPW * E
    assert planes % _NBUF == 0

    mesh = plsc.VectorSubcoreMesh(core_axis_name="c", subcore_axis_name="s")

    @functools.partial(
        pl.kernel,
        mesh=mesh,
        out_type=jax.ShapeDtypeStruct((planes * _N * E,), jnp.float32),
        scratch_types=[
            pltpu.VMEM((_RPW * SD,), jnp.float32),
            pltpu.VMEM((T * TD,), jnp.float32),
            [pltpu.VMEM((chunk,), jnp.float32) for _ in range(_NBUF)],
            [pltpu.SemaphoreType.DMA for _ in range(_NBUF)],
            [pltpu.SemaphoreType.DMA for _ in range(_NBUF)],
        ],
    )
    def sc_add_pe(x_hbm, ps_hbm, pt_hbm, out_hbm, ps_v, pt_v, bufs, isems, osems):
        wid = lax.axis_index("s") * 2 + lax.axis_index("c")
        pltpu.sync_copy(ps_hbm.at[pl.ds(wid * (_RPW * SD), _RPW * SD)], ps_v)
        pltpu.sync_copy(pt_hbm, pt_v)

        def base(p):
            return (p * _N + wid * _RPW) * E

        def start_in(p, b):
            pltpu.async_copy(x_hbm.at[pl.ds(base(p), chunk)], bufs[b], isems[b])

        def wait_in(b):
            pltpu.make_async_copy(x_hbm.at[pl.ds(0, chunk)], bufs[b], isems[b]).wait()

        def start_out(p, b):
            pltpu.async_copy(bufs[b], out_hbm.at[pl.ds(base(p), chunk)], osems[b])

        def wait_out(b):
            pltpu.make_async_copy(bufs[b], out_hbm.at[pl.ds(0, chunk)], osems[b]).wait()

        def compute(b, t):
            tb = t * TD
            buf_v = bufs[b]
            pt_vecs = tuple(pt_v[pl.ds(tb + c * 16, 16)] for c in range(TD // 16))

            def row_body(i, pt_c):
                boff = i * E
                poff = i * SD
                vs = [ps_v[pl.ds(poff + c * 16, 16)] for c in range(SD // 16)]
                for c in range(SD // 16):
                    plsc.addupdate(buf_v.at[pl.ds(boff + c * 16, 16)], vs[c])
                for c in range(TD // 16):
                    plsc.addupdate(buf_v.at[pl.ds(boff + SD + c * 16, 16)], pt_c[c])
                return pt_c

            lax.fori_loop(0, _RPW, row_body, pt_vecs)

        start_in(0, 0)
        start_in(1, 1)

        def quad_body(q, carry):
            for j in range(_NBUF):
                p = q * _NBUF + j
                wait_in(j)
                compute(j, lax.rem(p, T))
                start_out(p, j)
                nb = (j + 2) % _NBUF
                nxt = p + 2

                @pl.when(nxt < planes)
                def _():
                    @pl.when(p >= 2)
                    def _():
                        wait_out(nb)
                    start_in(nxt, nb)
            return carry

        lax.fori_loop(0, planes // _NBUF, quad_body, 0)
        for b in range(_NBUF):
            wait_out(b)

    return sc_add_pe


def kernel(x):
    B, T, N, E = x.shape
    ps_flat, pt_flat = _pe_tables(T)
    sc_add_pe = _make_sc_kernel(B * T, T)
    out = sc_add_pe(x.reshape(-1), ps_flat, pt_flat)
    return out.reshape(B, T, N, E)


# speedup vs baseline: 4.6897x; 4.6897x over previous
import math
import functools

import numpy as np
import jax
import jax.numpy as jnp
from jax.experimental import pallas as pl

_GRID = 32
_MAX_FRAMES = 32
_EMBED_DIM = 384
_SPATIAL_DIM = _EMBED_DIM * 2 // 3
_TEMPORAL_DIM = _EMBED_DIM - _SPATIAL_DIM
_X_DIM = _SPATIAL_DIM // 2
_Y_DIM = _SPATIAL_DIM - _X_DIM


def _create_pe_np(max_len, d):
    pos = np.arange(max_len, dtype=np.float32)[:, None]
    pe = np.zeros((max_len, d), dtype=np.float32)
    num_even = (d + 1) // 2
    num_odd = d // 2
    div_even = np.exp(np.arange(num_even, dtype=np.float32) * 2.0 * (-math.log(10000.0) / d))
    pe[:, 0::2] = np.sin(pos * div_even)
    if num_odd > 0:
        div_odd = np.exp(np.arange(num_odd, dtype=np.float32) * 2.0 * (-math.log(10000.0) / d))
        pe[:, 1::2] = np.cos(pos * div_odd)
    return pe


@functools.lru_cache(maxsize=None)
def _pe_tables(T):
    pe_x_tab = _create_pe_np(_GRID, _X_DIM)
    pe_y_tab = _create_pe_np(_GRID, _Y_DIM)
    pe_t_tab = _create_pe_np(_MAX_FRAMES, _TEMPORAL_DIM)
    yy, xx = np.meshgrid(np.arange(_GRID), np.arange(_GRID), indexing="ij")
    pe_x = pe_x_tab[xx.flatten()]
    pe_y = pe_y_tab[yy.flatten()]
    pe_spatial = np.concatenate([pe_x, pe_y], axis=-1)
    pe_t = pe_t_tab[:T][:, None, :]
    return jnp.asarray(pe_spatial), jnp.asarray(pe_t)


_TBLK = 8


def _add_pe_body(x_ref, ps_ref, pt_ref, o_ref):
    for i in range(_TBLK):
        xv = x_ref[0, i]
        o_ref[0, i, :, :_SPATIAL_DIM] = xv[:, :_SPATIAL_DIM] + ps_ref[...]
        o_ref[0, i, :, _SPATIAL_DIM:] = xv[:, _SPATIAL_DIM:] + pt_ref[i]


def kernel(x):
    B, T, N, E = x.shape
    pe_spatial, pe_t = _pe_tables(T)
    return pl.pallas_call(
        _add_pe_body,
        grid=(B, T // _TBLK),
        in_specs=[
            pl.BlockSpec((1, _TBLK, N, E), lambda b, t: (b, t, 0, 0)),
            pl.BlockSpec((N, _SPATIAL_DIM), lambda b, t: (0, 0)),
            pl.BlockSpec((_TBLK, 1, _TEMPORAL_DIM), lambda b, t: (t, 0, 0)),
        ],
        out_specs=pl.BlockSpec((1, _TBLK, N, E), lambda b, t: (b, t, 0, 0)),
        out_shape=jax.ShapeDtypeStruct((B, T, N, E), x.dtype),
    )(x, pe_spatial, pe_t)
